# baseline (speedup 1.0000x reference)
import functools

import jax
import jax.numpy as jnp
from jax import lax
from jax.experimental import pallas as pl
from jax.experimental.pallas import tpu as pltpu
from jax.experimental.pallas import tpu_sc as plsc

M, G, DZ = 16, 16384, 128
NT, DY = 8192, 64
B = M * NT

NC, NS = 2, 16
NW = NC * NS
CH = 128
NBUF = 4

SLICE_ROWS = (2, 4, 4, 4, 2)
SLICE_START = tuple(sum(SLICE_ROWS[:s]) for s in range(len(SLICE_ROWS)))
NSLICE = len(SLICE_ROWS)


def _sc_gather(slice_idx):
    ms = SLICE_ROWS[slice_idx]
    row0 = SLICE_START[slice_idx]
    b_per_w = ms * NT // NW
    w_per_row = NT // b_per_w
    nchunk = b_per_w // CH
    mesh = plsc.VectorSubcoreMesh(core_axis_name="c", subcore_axis_name="s")

    @functools.partial(
        pl.kernel,
        mesh=mesh,
        out_type=jax.ShapeDtypeStruct((ms, NT, DZ), jnp.float32),
        scratch_types=[
            pltpu.VMEM((b_per_w,), jnp.int32),
            *[pltpu.VMEM((CH, DZ), jnp.float32) for _ in range(NBUF)],
            *[pltpu.SemaphoreType.DMA for _ in range(2 * NBUF)],
        ],
    )
    def gather(table_hbm, mt_hbm, out_hbm, idx_v, *bufs_and_sems):
        rows = bufs_and_sems[:NBUF]
        gsem = bufs_and_sems[NBUF : 2 * NBUF]
        ssem = bufs_and_sems[2 * NBUF :]
        wid = lax.axis_index("s") * NC + lax.axis_index("c")
        iloc = wid // w_per_row
        i = row0 + iloc
        h = wid % w_per_row
        col0 = h * b_per_w
        pltpu.sync_copy(mt_hbm.at[i, pl.ds(col0, b_per_w)], idx_v)

        def gather_chunk(c, b):
            return pltpu.async_copy(
                table_hbm.at[i].at[idx_v.at[pl.ds(c * CH, CH)]], rows[b], gsem[b]
            )

        gcp = [None] * NBUF
        scp = [None] * NBUF
        gcp[0] = gather_chunk(0, 0)
        for c in range(nchunk):
            b = c % NBUF
            nb = (c + 1) % NBUF
            if c + 1 < nchunk:
                if scp[nb] is not None:
                    scp[nb].wait()
                gcp[nb] = gather_chunk(c + 1, nb)
            gcp[b].wait()
            scp[b] = pltpu.async_copy(
                rows[b], out_hbm.at[iloc, pl.ds(col0 + c * CH, CH)], ssem[b]
            )
        for b in range(NBUF):
            if scp[b] is not None:
                scp[b].wait()

    return gather


_gather_fns = [_sc_gather(s) for s in range(NSLICE)]


def _mm_body(zt_ref, wt_ref, b_ref, o_ref):
    o_ref[0] = (
        lax.dot_general(
            wt_ref[...],
            zt_ref[0],
            (((1,), (1,)), ((), ())),
            preferred_element_type=jnp.float32,
        )
        + b_ref[...]
    )


def _tc_linear_slice(slice_idx, zt_s, Wt, b2, out_prev):
    BN = 8192
    ms = SLICE_ROWS[slice_idx]
    row0 = SLICE_START[slice_idx]
    args = [zt_s, Wt, b2]
    in_specs = [
        pl.BlockSpec((1, BN, DZ), lambda i, j: (i, j, 0)),
        pl.BlockSpec((DY, DZ), lambda i, j: (0, 0)),
        pl.BlockSpec((DY, 1), lambda i, j: (0, 0)),
    ]
    aliases = {}
    if out_prev is not None:
        args.append(out_prev)
        in_specs.append(pl.BlockSpec(memory_space=pl.ANY))
        aliases = {3: 0}

    def body(zt_ref, wt_ref, b_ref, *rest):
        _mm_body(zt_ref, wt_ref, b_ref, rest[-1])

    return pl.pallas_call(
        body,
        grid=(ms, NT // BN),
        in_specs=in_specs,
        out_specs=pl.BlockSpec((1, DY, BN), lambda i, j: (row0 + i, 0, j)),
        out_shape=jax.ShapeDtypeStruct((M, DY, NT), jnp.float32),
        input_output_aliases=aliases,
    )(*args)


@jax.jit
def kernel(z_grid, mt, W, b):
    mt32 = mt.astype(jnp.int32)
    Wt = W.T
    b2 = b.reshape(DY, 1)
    zts = [_gather_fns[s](z_grid, mt32) for s in range(NSLICE)]
    out_t = None
    for s in range(NSLICE):
        out_t = _tc_linear_slice(s, zts[s], Wt, b2, out_t)
    return jnp.transpose(out_t, (0, 2, 1))


# speedup vs baseline: 1.0161x; 1.0161x over previous
import functools

import jax
import jax.numpy as jnp
from jax import lax
from jax.experimental import pallas as pl
from jax.experimental.pallas import tpu as pltpu
from jax.experimental.pallas import tpu_sc as plsc

M, G, DZ = 16, 16384, 128
NT, DY = 8192, 64
B = M * NT

NC, NS = 2, 16
NW = NC * NS
CH = 128
NBUF = 6

SLICE_ROWS = (4, 4, 4, 4)
SLICE_START = tuple(sum(SLICE_ROWS[:s]) for s in range(len(SLICE_ROWS)))
NSLICE = len(SLICE_ROWS)


def _sc_gather(slice_idx):
    ms = SLICE_ROWS[slice_idx]
    row0 = SLICE_START[slice_idx]
    b_per_w = ms * NT // NW
    w_per_row = NT // b_per_w
    nchunk = b_per_w // CH
    mesh = plsc.VectorSubcoreMesh(core_axis_name="c", subcore_axis_name="s")

    @functools.partial(
        pl.kernel,
        mesh=mesh,
        out_type=jax.ShapeDtypeStruct((ms, NT, DZ), jnp.float32),
        scratch_types=[
            pltpu.VMEM((b_per_w,), jnp.int32),
            *[pltpu.VMEM((CH, DZ), jnp.float32) for _ in range(NBUF)],
            *[pltpu.SemaphoreType.DMA for _ in range(2 * NBUF)],
        ],
    )
    def gather(table_hbm, mt_hbm, out_hbm, idx_v, *bufs_and_sems):
        rows = bufs_and_sems[:NBUF]
        gsem = bufs_and_sems[NBUF : 2 * NBUF]
        ssem = bufs_and_sems[2 * NBUF :]
        wid = lax.axis_index("s") * NC + lax.axis_index("c")
        iloc = wid // w_per_row
        i = row0 + iloc
        h = wid % w_per_row
        col0 = h * b_per_w
        pltpu.sync_copy(mt_hbm.at[i, pl.ds(col0, b_per_w)], idx_v)

        def gather_chunk(c, b):
            return pltpu.async_copy(
                table_hbm.at[i].at[idx_v.at[pl.ds(c * CH, CH)]], rows[b], gsem[b]
            )

        gcp = [None] * NBUF
        scp = [None] * NBUF
        gcp[0] = gather_chunk(0, 0)
        for c in range(nchunk):
            b = c % NBUF
            nb = (c + 1) % NBUF
            if c + 1 < nchunk:
                if scp[nb] is not None:
                    scp[nb].wait()
                gcp[nb] = gather_chunk(c + 1, nb)
            gcp[b].wait()
            scp[b] = pltpu.async_copy(
                rows[b], out_hbm.at[iloc, pl.ds(col0 + c * CH, CH)], ssem[b]
            )
        for b in range(NBUF):
            if scp[b] is not None:
                scp[b].wait()

    return gather


_gather_fns = [_sc_gather(s) for s in range(NSLICE)]


def _mm_body(zt_ref, wt_ref, b_ref, o_ref):
    o_ref[0] = (
        lax.dot_general(
            wt_ref[...],
            zt_ref[0],
            (((1,), (1,)), ((), ())),
            preferred_element_type=jnp.float32,
        )
        + b_ref[...]
    )


def _tc_linear_slice(slice_idx, zt_s, Wt, b2, out_prev):
    BN = 8192
    ms = SLICE_ROWS[slice_idx]
    row0 = SLICE_START[slice_idx]
    args = [zt_s, Wt, b2]
    in_specs = [
        pl.BlockSpec((1, BN, DZ), lambda i, j: (i, j, 0)),
        pl.BlockSpec((DY, DZ), lambda i, j: (0, 0)),
        pl.BlockSpec((DY, 1), lambda i, j: (0, 0)),
    ]
    aliases = {}
    if out_prev is not None:
        args.append(out_prev)
        in_specs.append(pl.BlockSpec(memory_space=pl.ANY))
        aliases = {3: 0}

    def body(zt_ref, wt_ref, b_ref, *rest):
        _mm_body(zt_ref, wt_ref, b_ref, rest[-1])

    return pl.pallas_call(
        body,
        grid=(ms, NT // BN),
        in_specs=in_specs,
        out_specs=pl.BlockSpec((1, DY, BN), lambda i, j: (row0 + i, 0, j)),
        out_shape=jax.ShapeDtypeStruct((M, DY, NT), jnp.float32),
        input_output_aliases=aliases,
    )(*args)


@jax.jit
def kernel(z_grid, mt, W, b):
    mt32 = mt.astype(jnp.int32)
    Wt = W.T
    b2 = b.reshape(DY, 1)
    zts = [_gather_fns[s](z_grid, mt32) for s in range(NSLICE)]
    out_t = None
    for s in range(NSLICE):
        out_t = _tc_linear_slice(s, zts[s], Wt, b2, out_t)
    return jnp.transpose(out_t, (0, 2, 1))


# speedup vs baseline: 1.0264x; 1.0101x over previous
import functools

import jax
import jax.numpy as jnp
from jax import lax
from jax.experimental import pallas as pl
from jax.experimental.pallas import tpu as pltpu
from jax.experimental.pallas import tpu_sc as plsc

M, G, DZ = 16, 16384, 128
NT, DY = 8192, 64
B = M * NT

NC, NS = 2, 16
NW = NC * NS
CH = 128
NBUF = 4

SLICE_ROWS = (4, 4, 4, 4)
SLICE_START = tuple(sum(SLICE_ROWS[:s]) for s in range(len(SLICE_ROWS)))
NSLICE = len(SLICE_ROWS)


def _sc_gather(slice_idx):
    ms = SLICE_ROWS[slice_idx]
    row0 = SLICE_START[slice_idx]
    b_per_w = ms * NT // NW
    w_per_row = NT // b_per_w
    nchunk = b_per_w // CH
    mesh = plsc.VectorSubcoreMesh(core_axis_name="c", subcore_axis_name="s")

    @functools.partial(
        pl.kernel,
        mesh=mesh,
        out_type=jax.ShapeDtypeStruct((ms, NT, DZ), jnp.float32),
        scratch_types=[
            pltpu.VMEM((b_per_w,), jnp.int32),
            *[pltpu.VMEM((CH, DZ), jnp.float32) for _ in range(NBUF)],
            *[pltpu.SemaphoreType.DMA for _ in range(2 * NBUF)],
        ],
    )
    def gather(table_hbm, mt_hbm, out_hbm, idx_v, *bufs_and_sems):
        rows = bufs_and_sems[:NBUF]
        gsem = bufs_and_sems[NBUF : 2 * NBUF]
        ssem = bufs_and_sems[2 * NBUF :]
        wid = lax.axis_index("s") * NC + lax.axis_index("c")
        iloc = wid // w_per_row
        i = row0 + iloc
        h = wid % w_per_row
        col0 = h * b_per_w
        pltpu.sync_copy(mt_hbm.at[i, pl.ds(col0, b_per_w)], idx_v)

        def gather_chunk(c, b):
            return pltpu.async_copy(
                table_hbm.at[i].at[idx_v.at[pl.ds(c * CH, CH)]], rows[b], gsem[b]
            )

        gcp = [None] * NBUF
        scp = [None] * NBUF
        gcp[0] = gather_chunk(0, 0)
        for c in range(nchunk):
            b = c % NBUF
            nb = (c + 1) % NBUF
            if c + 1 < nchunk:
                if scp[nb] is not None:
                    scp[nb].wait()
                gcp[nb] = gather_chunk(c + 1, nb)
            gcp[b].wait()
            scp[b] = pltpu.async_copy(
                rows[b], out_hbm.at[iloc, pl.ds(col0 + c * CH, CH)], ssem[b]
            )
        for b in range(NBUF):
            if scp[b] is not None:
                scp[b].wait()

    return gather


_gather_fns = [_sc_gather(s) for s in range(NSLICE)]


def _mm_body(zt_ref, wt_ref, b_ref, o_ref):
    o_ref[0] = (
        lax.dot_general(
            wt_ref[...],
            zt_ref[0],
            (((1,), (1,)), ((), ())),
            preferred_element_type=jnp.float32,
        )
        + b_ref[...]
    )


def _tc_linear_slice(slice_idx, zt_s, Wt, b2, out_prev):
    BN = 8192
    ms = SLICE_ROWS[slice_idx]
    row0 = SLICE_START[slice_idx]
    args = [zt_s, Wt, b2]
    in_specs = [
        pl.BlockSpec((1, BN, DZ), lambda i, j: (i, j, 0)),
        pl.BlockSpec((DY, DZ), lambda i, j: (0, 0)),
        pl.BlockSpec((DY, 1), lambda i, j: (0, 0)),
    ]
    aliases = {}
    if out_prev is not None:
        args.append(out_prev)
        in_specs.append(pl.BlockSpec(memory_space=pl.ANY))
        aliases = {3: 0}

    def body(zt_ref, wt_ref, b_ref, *rest):
        _mm_body(zt_ref, wt_ref, b_ref, rest[-1])

    return pl.pallas_call(
        body,
        grid=(ms, NT // BN),
        in_specs=in_specs,
        out_specs=pl.BlockSpec((1, DY, BN), lambda i, j: (row0 + i, 0, j)),
        out_shape=jax.ShapeDtypeStruct((M, DY, NT), jnp.float32),
        input_output_aliases=aliases,
    )(*args)


@jax.jit
def kernel(z_grid, mt, W, b):
    mt32 = mt.astype(jnp.int32)
    Wt = W.T
    b2 = b.reshape(DY, 1)
    zts = [_gather_fns[s](z_grid, mt32) for s in range(NSLICE)]
    out_t = None
    for s in range(NSLICE):
        out_t = _tc_linear_slice(s, zts[s], Wt, b2, out_t)
    return jnp.transpose(out_t, (0, 2, 1))
